# baseline (speedup 1.0000x reference)
import functools

import jax
import jax.numpy as jnp
from jax import lax
from jax.experimental import pallas as pl
from jax.experimental.pallas import tpu as pltpu
from jax.experimental.pallas import tpu_sc as plsc

NUM_TYPES = 8
DATA_SIZE = 10000
IN_DIM = 256
DIM = 128
BATCH = 16384

NUM_WORKERS = 32
PER_WORKER = BATCH // NUM_WORKERS
CHUNK = 128
NUM_CHUNKS = PER_WORKER // CHUNK

TC_BLOCK = 2048
NUM_TC_BLOCKS = BATCH // TC_BLOCK


def _sc_gather(codes, indices, tables_flat, lookup_table, n, base_off):
    mesh = plsc.VectorSubcoreMesh(core_axis_name="c", subcore_axis_name="s")
    per_worker = n // NUM_WORKERS
    num_chunks = per_worker // CHUNK

    @functools.partial(
        pl.kernel,
        out_type=(
            jax.ShapeDtypeStruct((n, IN_DIM), jnp.float32),
            jax.ShapeDtypeStruct((n, DIM), jnp.float32),
            jax.ShapeDtypeStruct((n,), jnp.int32),
        ),
        mesh=mesh,
        scratch_types=[
            pltpu.VMEM((per_worker,), jnp.int32),
            pltpu.VMEM((per_worker,), jnp.int32),
            pltpu.VMEM((per_worker,), jnp.int32),
            pltpu.VMEM((CHUNK, IN_DIM), jnp.float32),
            pltpu.VMEM((CHUNK, IN_DIM), jnp.float32),
            pltpu.VMEM((CHUNK, DIM), jnp.float32),
            pltpu.VMEM((CHUNK, DIM), jnp.float32),
            pltpu.SemaphoreType.DMA,
            pltpu.SemaphoreType.DMA,
            pltpu.SemaphoreType.DMA,
            pltpu.SemaphoreType.DMA,
            pltpu.SemaphoreType.DMA,
        ],
    )
    def sc_kernel(codes_hbm, idx_hbm, tab_hbm, lut_hbm,
                  emb_out, lb_out, code_out,
                  idx_v, c_v, row_v, emb_v0, emb_v1, lb_v0, lb_v1,
                  csem, gsem_e, gsem_l, wsem_e, wsem_l):
        wid = lax.axis_index("s") * 2 + lax.axis_index("c")
        base = wid * per_worker
        gbase = base_off + base
        pltpu.sync_copy(idx_hbm.at[pl.ds(gbase, per_worker)], idx_v)
        code_gets = [
            pltpu.async_copy(
                codes_hbm.at[idx_v.at[pl.ds(k * CHUNK, CHUNK)]],
                c_v.at[pl.ds(k * CHUNK, CHUNK)], csem)
            for k in range(num_chunks)
        ]
        for cg in code_gets:
            cg.wait()

        @pl.loop(0, per_worker, step=16)
        def _(i):
            c = c_v[pl.ds(i, 16)]
            pos = lax.iota(jnp.int32, 16) + (gbase + i)
            row_v[pl.ds(i, 16)] = jnp.where(c >= 0, c, pos)

        code_put = pltpu.async_copy(c_v, code_out.at[pl.ds(base, per_worker)],
                                    csem)

        emb_bufs = (emb_v0, emb_v1)
        lb_bufs = (lb_v0, lb_v1)

        def fire(k):
            e, l = emb_bufs[k % 2], lb_bufs[k % 2]
            ge = pltpu.async_copy(
                tab_hbm.at[row_v.at[pl.ds(k * CHUNK, CHUNK)]], e, gsem_e)
            gl = pltpu.async_copy(
                lut_hbm.at[idx_v.at[pl.ds(k * CHUNK, CHUNK)]], l, gsem_l)
            return ge, gl

        pend = fire(0)
        writes = [None] * num_chunks
        for k in range(num_chunks):
            if k >= 1:
                writes[k - 1][0].wait()
                writes[k - 1][1].wait()
            nxt = fire(k + 1) if k + 1 < num_chunks else None
            pend[0].wait()
            pend[1].wait()
            e, l = emb_bufs[k % 2], lb_bufs[k % 2]
            off = base + k * CHUNK
            writes[k] = (
                pltpu.async_copy(e, emb_out.at[pl.ds(off, CHUNK)], wsem_e),
                pltpu.async_copy(l, lb_out.at[pl.ds(off, CHUNK)], wsem_l),
            )
            pend = nxt
        writes[-1][0].wait()
        writes[-1][1].wait()
        code_put.wait()

    return sc_kernel(codes, indices, tables_flat, lookup_table)


def _tc_combine(codes_b, emb, lb, W, b, n, block_off, prev=None):
    num_blocks = n // TC_BLOCK
    def body(c_ref, emb_ref, lb_ref, w_ref, b_ref, o_ref):
        crow = c_ref[0]
        srow = jnp.where(crow >= 0, crow // DATA_SIZE, NUM_TYPES)
        scol = srow.reshape(TC_BLOCK, 1).astype(jnp.int16)
        sfull = scol + jnp.zeros((TC_BLOCK, DIM), jnp.int16)
        emb16 = emb_ref[...].astype(jnp.bfloat16)
        acc = lb_ref[...].astype(jnp.bfloat16)
        for tt in range(NUM_TYPES):
            prod = jnp.dot(emb16, w_ref[tt].astype(jnp.bfloat16),
                           preferred_element_type=jnp.float32)
            prod16 = prod.astype(jnp.bfloat16) + b_ref[tt].astype(jnp.bfloat16)[None, :]
            acc = jnp.where(sfull == jnp.int16(tt), prod16, acc)
        o_ref[...] = acc.astype(jnp.float32)

    in_specs = [
        pl.BlockSpec((1, 1, TC_BLOCK), lambda i: (i, 0, 0)),
        pl.BlockSpec((TC_BLOCK, IN_DIM), lambda i: (i, 0)),
        pl.BlockSpec((TC_BLOCK, DIM), lambda i: (i, 0)),
        pl.BlockSpec((NUM_TYPES, IN_DIM, DIM), lambda i: (0, 0, 0)),
        pl.BlockSpec((NUM_TYPES, DIM), lambda i: (0, 0)),
    ]
    args = [codes_b, emb, lb, W, b]
    aliases = {}
    if prev is not None:
        def body_alias(c_ref, emb_ref, lb_ref, w_ref, b_ref, prev_ref, o_ref):
            body(c_ref, emb_ref, lb_ref, w_ref, b_ref, o_ref)
        fn = body_alias
        in_specs = in_specs + [pl.BlockSpec(memory_space=pl.ANY)]
        args = args + [prev]
        aliases = {5: 0}
    else:
        fn = body
    return pl.pallas_call(
        fn,
        grid=(num_blocks,),
        in_specs=in_specs,
        out_specs=pl.BlockSpec((TC_BLOCK, DIM),
                               lambda i: (i + block_off, 0)),
        out_shape=jax.ShapeDtypeStruct((BATCH, DIM), jnp.float32),
        input_output_aliases=aliases,
        compiler_params=pltpu.CompilerParams(
            dimension_semantics=("arbitrary",),
        ),
    )(*args)


def kernel(indices, entity_types, entity_data_idx, tables, W, b, lookup_table):
    indices = indices.astype(jnp.int32)
    entity_types = entity_types.astype(jnp.int32)
    entity_data_idx = entity_data_idx.astype(jnp.int32)
    codes = jnp.where(entity_types < NUM_TYPES,
                      entity_types * DATA_SIZE + entity_data_idx,
                      -1).astype(jnp.int32)
    tables_flat = tables.reshape(NUM_TYPES * DATA_SIZE, IN_DIM)

    num_slices = 4
    part = BATCH // num_slices
    out = None
    for h in range(num_slices):
        emb, lb, codes_b = _sc_gather(codes, indices, tables_flat,
                                      lookup_table, part, h * part)
        out = _tc_combine(codes_b.reshape(part // TC_BLOCK, 1, TC_BLOCK),
                          emb, lb, W, b, part,
                          h * (part // TC_BLOCK), prev=out)
    return out


# speedup vs baseline: 1.0969x; 1.0969x over previous
import functools

import jax
import jax.numpy as jnp
from jax import lax
from jax.experimental import pallas as pl
from jax.experimental.pallas import tpu as pltpu
from jax.experimental.pallas import tpu_sc as plsc

NUM_TYPES = 8
DATA_SIZE = 10000
IN_DIM = 256
DIM = 128
BATCH = 16384

NUM_WORKERS = 32
PER_WORKER = BATCH // NUM_WORKERS
CHUNK = 128
NUM_CHUNKS = PER_WORKER // CHUNK

TC_BLOCK = 2048
NUM_TC_BLOCKS = BATCH // TC_BLOCK


def _sc_gather(codes, indices, tables_flat, lookup_table, n, base_off):
    mesh = plsc.VectorSubcoreMesh(core_axis_name="c", subcore_axis_name="s")
    per_worker = n // NUM_WORKERS
    num_chunks = per_worker // CHUNK

    @functools.partial(
        pl.kernel,
        out_type=(
            jax.ShapeDtypeStruct((n, IN_DIM), jnp.float32),
            jax.ShapeDtypeStruct((n, DIM), jnp.float32),
            jax.ShapeDtypeStruct((n,), jnp.int32),
        ),
        mesh=mesh,
        scratch_types=[
            pltpu.VMEM((per_worker,), jnp.int32),
            pltpu.VMEM((per_worker,), jnp.int32),
            pltpu.VMEM((per_worker,), jnp.int32),
            pltpu.VMEM((CHUNK, IN_DIM), jnp.float32),
            pltpu.VMEM((CHUNK, IN_DIM), jnp.float32),
            pltpu.VMEM((CHUNK, DIM), jnp.float32),
            pltpu.VMEM((CHUNK, DIM), jnp.float32),
            pltpu.SemaphoreType.DMA,
            pltpu.SemaphoreType.DMA,
            pltpu.SemaphoreType.DMA,
            pltpu.SemaphoreType.DMA,
            pltpu.SemaphoreType.DMA,
        ],
    )
    def sc_kernel(codes_hbm, idx_hbm, tab_hbm, lut_hbm,
                  emb_out, lb_out, code_out,
                  idx_v, c_v, row_v, emb_v0, emb_v1, lb_v0, lb_v1,
                  csem, gsem_e, gsem_l, wsem_e, wsem_l):
        wid = lax.axis_index("s") * 2 + lax.axis_index("c")
        base = wid * per_worker
        gbase = base_off + base
        pltpu.sync_copy(idx_hbm.at[pl.ds(gbase, per_worker)], idx_v)
        code_gets = [
            pltpu.async_copy(
                codes_hbm.at[idx_v.at[pl.ds(k * CHUNK, CHUNK)]],
                c_v.at[pl.ds(k * CHUNK, CHUNK)], csem)
            for k in range(num_chunks)
        ]
        for cg in code_gets:
            cg.wait()

        @pl.loop(0, per_worker, step=16)
        def _(i):
            c = c_v[pl.ds(i, 16)]
            pos = lax.iota(jnp.int32, 16) + (gbase + i)
            row_v[pl.ds(i, 16)] = jnp.where(c >= 0, c, pos)

        code_put = pltpu.async_copy(c_v, code_out.at[pl.ds(base, per_worker)],
                                    csem)

        emb_bufs = (emb_v0, emb_v1)
        lb_bufs = (lb_v0, lb_v1)

        def fire(k):
            e, l = emb_bufs[k % 2], lb_bufs[k % 2]
            ge = pltpu.async_copy(
                tab_hbm.at[row_v.at[pl.ds(k * CHUNK, CHUNK)]], e, gsem_e)
            gl = pltpu.async_copy(
                lut_hbm.at[idx_v.at[pl.ds(k * CHUNK, CHUNK)]], l, gsem_l)
            return ge, gl

        pend = fire(0)
        writes = [None] * num_chunks
        for k in range(num_chunks):
            if k >= 1:
                writes[k - 1][0].wait()
                writes[k - 1][1].wait()
            nxt = fire(k + 1) if k + 1 < num_chunks else None
            pend[0].wait()
            pend[1].wait()
            e, l = emb_bufs[k % 2], lb_bufs[k % 2]
            off = base + k * CHUNK
            writes[k] = (
                pltpu.async_copy(e, emb_out.at[pl.ds(off, CHUNK)], wsem_e),
                pltpu.async_copy(l, lb_out.at[pl.ds(off, CHUNK)], wsem_l),
            )
            pend = nxt
        writes[-1][0].wait()
        writes[-1][1].wait()
        code_put.wait()

    return sc_kernel(codes, indices, tables_flat, lookup_table)


def _tc_combine(codes_b, emb, lb, W, b, n, block_off, prev=None):
    num_blocks = n // TC_BLOCK
    def body(c_ref, emb_ref, lb_ref, w_ref, b_ref, o_ref):
        crow = c_ref[0]
        srow = jnp.where(crow >= 0, crow // DATA_SIZE, NUM_TYPES)
        scol = srow.reshape(TC_BLOCK, 1).astype(jnp.int16)
        sfull = scol + jnp.zeros((TC_BLOCK, DIM), jnp.int16)
        emb16 = emb_ref[...].astype(jnp.bfloat16)
        acc = lb_ref[...].astype(jnp.bfloat16)
        for tt in range(NUM_TYPES):
            prod = jnp.dot(emb16, w_ref[tt].astype(jnp.bfloat16),
                           preferred_element_type=jnp.float32)
            prod16 = prod.astype(jnp.bfloat16) + b_ref[tt].astype(jnp.bfloat16)[None, :]
            acc = jnp.where(sfull == jnp.int16(tt), prod16, acc)
        o_ref[...] = acc.astype(jnp.float32)

    in_specs = [
        pl.BlockSpec((1, 1, TC_BLOCK), lambda i: (i, 0, 0)),
        pl.BlockSpec((TC_BLOCK, IN_DIM), lambda i: (i, 0)),
        pl.BlockSpec((TC_BLOCK, DIM), lambda i: (i, 0)),
        pl.BlockSpec((NUM_TYPES, IN_DIM, DIM), lambda i: (0, 0, 0)),
        pl.BlockSpec((NUM_TYPES, DIM), lambda i: (0, 0)),
    ]
    args = [codes_b, emb, lb, W, b]
    aliases = {}
    if prev is not None:
        def body_alias(c_ref, emb_ref, lb_ref, w_ref, b_ref, prev_ref, o_ref):
            body(c_ref, emb_ref, lb_ref, w_ref, b_ref, o_ref)
        fn = body_alias
        in_specs = in_specs + [pl.BlockSpec(memory_space=pl.ANY)]
        args = args + [prev]
        aliases = {5: 0}
    else:
        fn = body
    return pl.pallas_call(
        fn,
        grid=(num_blocks,),
        in_specs=in_specs,
        out_specs=pl.BlockSpec((TC_BLOCK, DIM),
                               lambda i: (i + block_off, 0)),
        out_shape=jax.ShapeDtypeStruct((BATCH, DIM), jnp.float32),
        input_output_aliases=aliases,
        compiler_params=pltpu.CompilerParams(
            dimension_semantics=("arbitrary",),
        ),
    )(*args)


def kernel(indices, entity_types, entity_data_idx, tables, W, b, lookup_table):
    indices = indices.astype(jnp.int32)
    entity_types = entity_types.astype(jnp.int32)
    entity_data_idx = entity_data_idx.astype(jnp.int32)
    codes = jnp.where(entity_types < NUM_TYPES,
                      entity_types * DATA_SIZE + entity_data_idx,
                      -1).astype(jnp.int32)
    tables_flat = tables.reshape(NUM_TYPES * DATA_SIZE, IN_DIM)

    num_slices = 2
    part = BATCH // num_slices
    out = None
    for h in range(num_slices):
        emb, lb, codes_b = _sc_gather(codes, indices, tables_flat,
                                      lookup_table, part, h * part)
        out = _tc_combine(codes_b.reshape(part // TC_BLOCK, 1, TC_BLOCK),
                          emb, lb, W, b, part,
                          h * (part // TC_BLOCK), prev=out)
    return out


# speedup vs baseline: 1.1107x; 1.0126x over previous
import functools

import jax
import jax.numpy as jnp
from jax import lax
from jax.experimental import pallas as pl
from jax.experimental.pallas import tpu as pltpu
from jax.experimental.pallas import tpu_sc as plsc

NUM_TYPES = 8
DATA_SIZE = 10000
IN_DIM = 256
DIM = 128
BATCH = 16384

NUM_WORKERS = 32
PER_WORKER = BATCH // NUM_WORKERS
CHUNK = 128
NUM_CHUNKS = PER_WORKER // CHUNK

TC_BLOCK = 4096
NUM_TC_BLOCKS = BATCH // TC_BLOCK


def _sc_gather(codes, indices, tables_flat, lookup_table, n, base_off):
    mesh = plsc.VectorSubcoreMesh(core_axis_name="c", subcore_axis_name="s")
    per_worker = n // NUM_WORKERS
    num_chunks = per_worker // CHUNK

    @functools.partial(
        pl.kernel,
        out_type=(
            jax.ShapeDtypeStruct((n, IN_DIM), jnp.float32),
            jax.ShapeDtypeStruct((n, DIM), jnp.float32),
            jax.ShapeDtypeStruct((n,), jnp.int32),
        ),
        mesh=mesh,
        scratch_types=[
            pltpu.VMEM((per_worker,), jnp.int32),
            pltpu.VMEM((per_worker,), jnp.int32),
            pltpu.VMEM((per_worker,), jnp.int32),
            pltpu.VMEM((CHUNK, IN_DIM), jnp.float32),
            pltpu.VMEM((CHUNK, IN_DIM), jnp.float32),
            pltpu.VMEM((CHUNK, DIM), jnp.float32),
            pltpu.VMEM((CHUNK, DIM), jnp.float32),
            pltpu.SemaphoreType.DMA,
            pltpu.SemaphoreType.DMA,
            pltpu.SemaphoreType.DMA,
            pltpu.SemaphoreType.DMA,
            pltpu.SemaphoreType.DMA,
        ],
    )
    def sc_kernel(codes_hbm, idx_hbm, tab_hbm, lut_hbm,
                  emb_out, lb_out, code_out,
                  idx_v, c_v, row_v, emb_v0, emb_v1, lb_v0, lb_v1,
                  csem, gsem_e, gsem_l, wsem_e, wsem_l):
        wid = lax.axis_index("s") * 2 + lax.axis_index("c")
        base = wid * per_worker
        gbase = base_off + base
        pltpu.sync_copy(idx_hbm.at[pl.ds(gbase, per_worker)], idx_v)
        code_gets = [
            pltpu.async_copy(
                codes_hbm.at[idx_v.at[pl.ds(k * CHUNK, CHUNK)]],
                c_v.at[pl.ds(k * CHUNK, CHUNK)], csem)
            for k in range(num_chunks)
        ]
        for cg in code_gets:
            cg.wait()

        @pl.loop(0, per_worker, step=16)
        def _(i):
            c = c_v[pl.ds(i, 16)]
            pos = lax.iota(jnp.int32, 16) + (gbase + i)
            row_v[pl.ds(i, 16)] = jnp.where(c >= 0, c, pos)

        code_put = pltpu.async_copy(c_v, code_out.at[pl.ds(base, per_worker)],
                                    csem)

        emb_bufs = (emb_v0, emb_v1)
        lb_bufs = (lb_v0, lb_v1)

        def fire(k):
            e, l = emb_bufs[k % 2], lb_bufs[k % 2]
            ge = pltpu.async_copy(
                tab_hbm.at[row_v.at[pl.ds(k * CHUNK, CHUNK)]], e, gsem_e)
            gl = pltpu.async_copy(
                lut_hbm.at[idx_v.at[pl.ds(k * CHUNK, CHUNK)]], l, gsem_l)
            return ge, gl

        pend = fire(0)
        writes = [None] * num_chunks
        for k in range(num_chunks):
            if k >= 1:
                writes[k - 1][0].wait()
                writes[k - 1][1].wait()
            nxt = fire(k + 1) if k + 1 < num_chunks else None
            pend[0].wait()
            pend[1].wait()
            e, l = emb_bufs[k % 2], lb_bufs[k % 2]
            off = base + k * CHUNK
            writes[k] = (
                pltpu.async_copy(e, emb_out.at[pl.ds(off, CHUNK)], wsem_e),
                pltpu.async_copy(l, lb_out.at[pl.ds(off, CHUNK)], wsem_l),
            )
            pend = nxt
        writes[-1][0].wait()
        writes[-1][1].wait()
        code_put.wait()

    return sc_kernel(codes, indices, tables_flat, lookup_table)


def _tc_combine(codes_b, emb, lb, W, b, n, block_off, prev=None):
    num_blocks = n // TC_BLOCK
    def body(c_ref, emb_ref, lb_ref, w_ref, b_ref, o_ref):
        crow = c_ref[0]
        srow = jnp.where(crow >= 0, crow // DATA_SIZE, NUM_TYPES)
        scol = srow.reshape(TC_BLOCK, 1).astype(jnp.int16)
        sfull = scol + jnp.zeros((TC_BLOCK, DIM), jnp.int16)
        emb16 = emb_ref[...].astype(jnp.bfloat16)
        acc = lb_ref[...].astype(jnp.bfloat16)
        for tt in range(NUM_TYPES):
            prod = jnp.dot(emb16, w_ref[tt].astype(jnp.bfloat16),
                           preferred_element_type=jnp.float32)
            prod16 = prod.astype(jnp.bfloat16) + b_ref[tt].astype(jnp.bfloat16)[None, :]
            acc = jnp.where(sfull == jnp.int16(tt), prod16, acc)
        o_ref[...] = acc.astype(jnp.float32)

    in_specs = [
        pl.BlockSpec((1, 1, TC_BLOCK), lambda i: (i, 0, 0)),
        pl.BlockSpec((TC_BLOCK, IN_DIM), lambda i: (i, 0)),
        pl.BlockSpec((TC_BLOCK, DIM), lambda i: (i, 0)),
        pl.BlockSpec((NUM_TYPES, IN_DIM, DIM), lambda i: (0, 0, 0)),
        pl.BlockSpec((NUM_TYPES, DIM), lambda i: (0, 0)),
    ]
    args = [codes_b, emb, lb, W, b]
    aliases = {}
    if prev is not None:
        def body_alias(c_ref, emb_ref, lb_ref, w_ref, b_ref, prev_ref, o_ref):
            body(c_ref, emb_ref, lb_ref, w_ref, b_ref, o_ref)
        fn = body_alias
        in_specs = in_specs + [pl.BlockSpec(memory_space=pl.ANY)]
        args = args + [prev]
        aliases = {5: 0}
    else:
        fn = body
    return pl.pallas_call(
        fn,
        grid=(num_blocks,),
        in_specs=in_specs,
        out_specs=pl.BlockSpec((TC_BLOCK, DIM),
                               lambda i: (i + block_off, 0)),
        out_shape=jax.ShapeDtypeStruct((BATCH, DIM), jnp.float32),
        input_output_aliases=aliases,
        compiler_params=pltpu.CompilerParams(
            dimension_semantics=("arbitrary",),
        ),
    )(*args)


def kernel(indices, entity_types, entity_data_idx, tables, W, b, lookup_table):
    indices = indices.astype(jnp.int32)
    entity_types = entity_types.astype(jnp.int32)
    entity_data_idx = entity_data_idx.astype(jnp.int32)
    codes = jnp.where(entity_types < NUM_TYPES,
                      entity_types * DATA_SIZE + entity_data_idx,
                      -1).astype(jnp.int32)
    tables_flat = tables.reshape(NUM_TYPES * DATA_SIZE, IN_DIM)

    num_slices = 2
    part = BATCH // num_slices
    out = None
    for h in range(num_slices):
        emb, lb, codes_b = _sc_gather(codes, indices, tables_flat,
                                      lookup_table, part, h * part)
        out = _tc_combine(codes_b.reshape(part // TC_BLOCK, 1, TC_BLOCK),
                          emb, lb, W, b, part,
                          h * (part // TC_BLOCK), prev=out)
    return out
